# baseline (speedup 1.0000x reference)
import functools

import jax
import jax.numpy as jnp
from jax import lax
from jax.experimental import pallas as pl
from jax.experimental.pallas import tpu as pltpu
from jax.experimental.pallas import tpu_sc as plsc

B, T, D = 1, 2048, 1024
H = 16
HD = D // H
HALF = HD // 2
E = 8
K = 2
INTER = 1024
SCALE = D ** (-0.5)

TQ = 256
TILE = 128
NTILES = (T * K + E * (TILE - 1)) // TILE + 1
NPOS = NTILES * TILE

NC, NS = 2, 16
NW = NC * NS
GCH = 32
CCH = 16


def _prelude_body(x_ref, g1_ref, wq_ref, bq_ref, qkv_ref, cos_ref, sin_ref):
    i = pl.program_id(0)
    xb = x_ref[...]
    ms = jnp.mean(xb * xb, axis=-1, keepdims=True)
    xn = xb * lax.rsqrt(ms + 1e-6) * g1_ref[...]
    qkv = lax.dot_general(xn, wq_ref[...], (((1,), (1,)), ((), ())),
                          preferred_element_type=jnp.float32)
    qkv_ref[...] = qkv + bq_ref[...]
    pos = (i * TQ
           + lax.broadcasted_iota(jnp.int32, (TQ, HALF), 0)).astype(jnp.float32)
    expnt = lax.broadcasted_iota(
        jnp.int32, (TQ, HALF), 1).astype(jnp.float32) / HALF
    freq = pos * jnp.exp(expnt * (-jnp.log(10000.0)))
    cos_ref[...] = jnp.cos(freq)
    sin_ref[...] = jnp.sin(freq)


def _attn_body(q_ref, k_ref, v_ref, cq_ref, sq_ref, ck_ref, sk_ref, o_ref):
    qi = pl.program_id(1)
    q = q_ref[0]
    k = k_ref[0]
    v = v_ref[0]
    cq, sq = cq_ref[...], sq_ref[...]
    ck, sk = ck_ref[...], sk_ref[...]
    q1, q2 = q[:, :HALF], q[:, HALF:]
    qr = jnp.concatenate([q1 * cq - q2 * sq, q1 * sq + q2 * cq], axis=-1)
    qr = qr * SCALE
    k1, k2 = k[:, :HALF], k[:, HALF:]
    kr = jnp.concatenate([k1 * ck - k2 * sk, k1 * sk + k2 * ck], axis=-1)
    s = lax.dot_general(qr, kr, (((1,), (1,)), ((), ())),
                        preferred_element_type=jnp.float32)
    rows = qi * TQ + lax.broadcasted_iota(jnp.int32, (TQ, T), 0)
    cols = lax.broadcasted_iota(jnp.int32, (TQ, T), 1)
    s = jnp.where(cols > rows, -1e9, s)
    m = jnp.max(s, axis=-1, keepdims=True)
    p = jnp.exp(s - m)
    w = p / jnp.sum(p, axis=-1, keepdims=True)
    o_ref[0] = lax.dot_general(w, v, (((1,), (0,)), ((), ())),
                               preferred_element_type=jnp.float32)


def _post_body(a_ref, x_ref, wo_ref, bo_ref, g2_ref, gw_ref,
               x1_ref, xn2_ref, rt_ref):
    a = a_ref[...]
    o = lax.dot_general(a, wo_ref[...], (((1,), (1,)), ((), ())),
                        preferred_element_type=jnp.float32)
    x1 = o + bo_ref[...] + x_ref[...]
    x1_ref[...] = x1
    ms = jnp.mean(x1 * x1, axis=-1, keepdims=True)
    xn2 = x1 * lax.rsqrt(ms + 1e-6) * g2_ref[...]
    xn2_ref[...] = xn2
    lg = lax.dot_general(xn2, gw_ref[...], (((1,), (1,)), ((), ())),
                         preferred_element_type=jnp.float32)
    mx = jnp.max(lg, axis=-1, keepdims=True)
    ex = jnp.exp(lg - mx)
    p = ex / jnp.sum(ex, axis=-1, keepdims=True)
    colsE = lax.broadcasted_iota(jnp.int32, (TQ, E), 1)
    v1 = jnp.max(p, axis=-1, keepdims=True)
    i1 = jnp.min(jnp.where(p == v1, colsE, E), axis=-1, keepdims=True)
    p2 = jnp.where(colsE == i1, -1.0, p)
    v2 = jnp.max(p2, axis=-1, keepdims=True)
    i2 = jnp.min(jnp.where(p2 == v2, colsE, E), axis=-1, keepdims=True)
    rt_ref[...] = jnp.concatenate(
        [v1, v2, i1.astype(jnp.float32), i2.astype(jnp.float32),
         jnp.zeros((TQ, 4), jnp.float32)], axis=-1)


def _ffn_body(eids_ref, xs_ref, w1_ref, w3_ref, w2_ref,
              b1_ref, b3_ref, b2_ref, wp_ref, o_ref):
    del eids_ref
    xb = xs_ref[...]
    h1 = lax.dot_general(xb, w1_ref[0], (((1,), (1,)), ((), ())),
                         preferred_element_type=jnp.float32) + b1_ref[0]
    h3 = lax.dot_general(xb, w3_ref[0], (((1,), (1,)), ((), ())),
                         preferred_element_type=jnp.float32) + b3_ref[0]
    h = (h1 * lax.logistic(h1)) * h3
    o = lax.dot_general(h, w2_ref[0], (((1,), (1,)), ((), ())),
                        preferred_element_type=jnp.float32) + b2_ref[0]
    o_ref[...] = o * wp_ref[:, 0:1]


def _sc_mesh():
    return plsc.VectorSubcoreMesh(core_axis_name="c", subcore_axis_name="s")


def _dispatch_gather(xn2, tok_for_pos):
    per_w = NPOS // NW

    @functools.partial(
        pl.kernel,
        out_type=jax.ShapeDtypeStruct((NPOS, D), jnp.float32),
        mesh=_sc_mesh(),
        scratch_types=[
            pltpu.VMEM((GCH,), jnp.int32),
            pltpu.VMEM((GCH, D), jnp.float32),
            pltpu.SemaphoreType.DMA,
        ],
    )
    def k(tab_hbm, idx_hbm, out_hbm, idx_v, rows_v, sem):
        wid = lax.axis_index("s") * NC + lax.axis_index("c")
        base = wid * per_w

        @pl.loop(0, per_w // GCH)
        def _(c):
            off = base + c * GCH
            pltpu.sync_copy(idx_hbm.at[pl.ds(off, GCH)], idx_v)
            pltpu.async_copy(tab_hbm.at[idx_v], rows_v, sem).wait()
            pltpu.sync_copy(rows_v, out_hbm.at[pl.ds(off, GCH)])

    return k(xn2, tok_for_pos)


def _combine(ys, p0, p1, x1):
    per_w = T // NW

    @functools.partial(
        pl.kernel,
        out_type=jax.ShapeDtypeStruct((T, D), jnp.float32),
        mesh=_sc_mesh(),
        scratch_types=[
            pltpu.VMEM((CCH,), jnp.int32),
            pltpu.VMEM((CCH,), jnp.int32),
            pltpu.VMEM((CCH, D), jnp.float32),
            pltpu.VMEM((CCH, D), jnp.float32),
            pltpu.VMEM((CCH, D), jnp.float32),
            pltpu.SemaphoreType.DMA,
        ],
    )
    def k(ys_hbm, p0_hbm, p1_hbm, x1_hbm, out_hbm,
          i0_v, i1_v, a_v, b_v, c_v, sem):
        wid = lax.axis_index("s") * NC + lax.axis_index("c")
        base = wid * per_w

        @pl.loop(0, per_w // CCH)
        def _(c):
            off = base + c * CCH
            pltpu.sync_copy(p0_hbm.at[pl.ds(off, CCH)], i0_v)
            pltpu.sync_copy(p1_hbm.at[pl.ds(off, CCH)], i1_v)
            pltpu.async_copy(ys_hbm.at[i0_v], a_v, sem).wait()
            pltpu.async_copy(ys_hbm.at[i1_v], b_v, sem).wait()
            pltpu.sync_copy(x1_hbm.at[pl.ds(off, CCH)], c_v)

            @pl.loop(0, CCH)
            def _(r):
                @pl.loop(0, D, step=16)
                def _(cc):
                    sl = (r, pl.ds(cc, 16))
                    a_v[sl] = a_v[sl] + b_v[sl] + c_v[sl]

            pltpu.sync_copy(a_v, out_hbm.at[pl.ds(off, CCH)])

    return k(ys, p0, p1, x1)


def _routing_meta(route):
    vals = route[:, :K]
    idx = route[:, K:2 * K].astype(jnp.int32)
    e_flat = idx.reshape(-1)
    val_flat = vals.reshape(-1)
    oh = (e_flat[:, None] == jnp.arange(E, dtype=jnp.int32)[None, :])
    oh = oh.astype(jnp.int32)
    ranks = jnp.cumsum(oh, axis=0) - oh
    r = jnp.sum(ranks * oh, axis=1)
    counts = jnp.sum(oh, axis=0)
    padded = ((counts + TILE - 1) // TILE) * TILE
    offs = jnp.concatenate(
        [jnp.zeros((1,), padded.dtype), jnp.cumsum(padded)[:-1]])
    P = (offs[e_flat] + r).astype(jnp.int32)
    tok_flat = jnp.arange(T * K, dtype=jnp.int32) // K
    tok_for_pos = jnp.zeros((NPOS,), jnp.int32).at[P].set(tok_flat)
    w_pos = jnp.zeros((NPOS,), jnp.float32).at[P].set(val_flat)
    cum = jnp.cumsum(padded)
    tile_starts = jnp.arange(NTILES, dtype=cum.dtype) * TILE
    eids = jnp.minimum(
        jnp.searchsorted(cum, tile_starts, side='right'), E - 1)
    eids = eids.astype(jnp.int32)
    Ppair = P.reshape(T, K)
    return tok_for_pos, w_pos, eids, Ppair[:, 0], Ppair[:, 1]


def kernel(x, g1, g2, Wqkv, bqkv, Wout, bout, gateW, w1, b1, w2, b2, w3, b3):
    xf = x.reshape(T, D)
    Wq2 = Wqkv.reshape(H, 3, HD, D).transpose(1, 0, 2, 3).reshape(3 * D, D)
    bq2 = bqkv.reshape(H, 3, HD).transpose(1, 0, 2).reshape(1, 3 * D)

    qkv, cosf, sinf = pl.pallas_call(
        _prelude_body,
        grid=(T // TQ,),
        in_specs=[
            pl.BlockSpec((TQ, D), lambda i: (i, 0)),
            pl.BlockSpec((1, D), lambda i: (0, 0)),
            pl.BlockSpec((3 * D, D), lambda i: (0, 0)),
            pl.BlockSpec((1, 3 * D), lambda i: (0, 0)),
        ],
        out_specs=[
            pl.BlockSpec((TQ, 3 * D), lambda i: (i, 0)),
            pl.BlockSpec((TQ, HALF), lambda i: (i, 0)),
            pl.BlockSpec((TQ, HALF), lambda i: (i, 0)),
        ],
        out_shape=[
            jax.ShapeDtypeStruct((T, 3 * D), jnp.float32),
            jax.ShapeDtypeStruct((T, HALF), jnp.float32),
            jax.ShapeDtypeStruct((T, HALF), jnp.float32),
        ],
    )(xf, g1.reshape(1, D), Wq2, bq2)

    q3 = qkv[:, :D].reshape(T, H, HD).transpose(1, 0, 2)
    k3 = qkv[:, D:2 * D].reshape(T, H, HD).transpose(1, 0, 2)
    v3 = qkv[:, 2 * D:].reshape(T, H, HD).transpose(1, 0, 2)

    attn = pl.pallas_call(
        _attn_body,
        grid=(H, T // TQ),
        in_specs=[
            pl.BlockSpec((1, TQ, HD), lambda h, i: (h, i, 0)),
            pl.BlockSpec((1, T, HD), lambda h, i: (h, 0, 0)),
            pl.BlockSpec((1, T, HD), lambda h, i: (h, 0, 0)),
            pl.BlockSpec((TQ, HALF), lambda h, i: (i, 0)),
            pl.BlockSpec((TQ, HALF), lambda h, i: (i, 0)),
            pl.BlockSpec((T, HALF), lambda h, i: (0, 0)),
            pl.BlockSpec((T, HALF), lambda h, i: (0, 0)),
        ],
        out_specs=pl.BlockSpec((1, TQ, HD), lambda h, i: (h, i, 0)),
        out_shape=jax.ShapeDtypeStruct((H, T, HD), jnp.float32),
    )(q3, k3, v3, cosf, sinf, cosf, sinf)

    attn_t = attn.transpose(1, 0, 2).reshape(T, D)

    x1, xn2, route = pl.pallas_call(
        _post_body,
        grid=(T // TQ,),
        in_specs=[
            pl.BlockSpec((TQ, D), lambda i: (i, 0)),
            pl.BlockSpec((TQ, D), lambda i: (i, 0)),
            pl.BlockSpec((D, D), lambda i: (0, 0)),
            pl.BlockSpec((1, D), lambda i: (0, 0)),
            pl.BlockSpec((1, D), lambda i: (0, 0)),
            pl.BlockSpec((E, D), lambda i: (0, 0)),
        ],
        out_specs=[
            pl.BlockSpec((TQ, D), lambda i: (i, 0)),
            pl.BlockSpec((TQ, D), lambda i: (i, 0)),
            pl.BlockSpec((TQ, E), lambda i: (i, 0)),
        ],
        out_shape=[
            jax.ShapeDtypeStruct((T, D), jnp.float32),
            jax.ShapeDtypeStruct((T, D), jnp.float32),
            jax.ShapeDtypeStruct((T, E), jnp.float32),
        ],
    )(attn_t, xf, Wout, bout.reshape(1, D), g2.reshape(1, D), gateW)

    tok_for_pos, w_pos, eids, p0, p1 = _routing_meta(route)

    xs = _dispatch_gather(xn2, tok_for_pos)

    wp2 = jnp.broadcast_to(w_pos[:, None], (NPOS, 128))

    ys = pl.pallas_call(
        _ffn_body,
        grid_spec=pltpu.PrefetchScalarGridSpec(
            num_scalar_prefetch=1,
            grid=(NTILES,),
            in_specs=[
                pl.BlockSpec((TILE, D), lambda i, eids: (i, 0)),
                pl.BlockSpec((1, INTER, D), lambda i, eids: (eids[i], 0, 0)),
                pl.BlockSpec((1, INTER, D), lambda i, eids: (eids[i], 0, 0)),
                pl.BlockSpec((1, D, INTER), lambda i, eids: (eids[i], 0, 0)),
                pl.BlockSpec((1, 1, INTER), lambda i, eids: (eids[i], 0, 0)),
                pl.BlockSpec((1, 1, INTER), lambda i, eids: (eids[i], 0, 0)),
                pl.BlockSpec((1, 1, D), lambda i, eids: (eids[i], 0, 0)),
                pl.BlockSpec((TILE, 128), lambda i, eids: (i, 0)),
            ],
            out_specs=pl.BlockSpec((TILE, D), lambda i, eids: (i, 0)),
        ),
        out_shape=jax.ShapeDtypeStruct((NPOS, D), jnp.float32),
    )(eids, xs, w1, w3, w2,
      b1.reshape(E, 1, INTER), b3.reshape(E, 1, INTER), b2.reshape(E, 1, D),
      wp2)

    out = _combine(ys, p0, p1, x1)
    return out.reshape(B, T, D)
